# baseline (speedup 1.0000x reference)
import jax
import jax.numpy as jnp
from jax.experimental import pallas as pl
from jax.experimental.pallas import tpu as pltpu

_ROW_TILE = 256
_GROUP = 64
_STRIDE = _GROUP + 1
_LANES = 128
_N_ACC = 4


def _stats_kernel(table_ref, d_ref):
    x = table_ref[...]
    m = jnp.max(x, axis=-1, keepdims=True)
    s = jnp.sum(jnp.exp(x - m), axis=-1, keepdims=True)
    d_ref[...] = (jnp.log(s) + m) - x


def _make_main_kernel(tm, v_shift, chunks):
    def _main_kernel(flat_ref, table_ref, d_ref, out_ref, part_ref,
                     ts_a, ts_b):
        accs = [jnp.zeros((1, _LANES), jnp.float32) for _ in range(_N_ACC)]
        for g in range(tm // _GROUP):
            ts = ts_a if g % 2 == 0 else ts_b
            for mi in range(_GROUP):
                m = g * _GROUP + mi
                f = flat_ref[0, 0, m]
                i16 = pl.multiple_of((f >> v_shift) * chunks,
                                     8 if chunks % 8 == 0 else chunks)
                slab = table_ref[pl.ds(i16, chunks), :]
                ts[mi:mi + (chunks - 1) * _STRIDE + 1:_STRIDE, :] = slab
                chunk = d_ref[f >> 7]
                accs[m % _N_ACC] = accs[m % _N_ACC] + pltpu.roll(
                    chunk, -(f & (_LANES - 1)), axis=1)
            r0 = g * _GROUP
            for j in range(chunks):
                out_ref[r0:r0 + _GROUP, j * _LANES:(j + 1) * _LANES] = (
                    ts[j * _STRIDE:j * _STRIDE + _GROUP, :])
        acc = (accs[0] + accs[1]) + (accs[2] + accs[3])
        part_ref[...] = acc.reshape(1, 1, _LANES)
    return _main_kernel


def kernel(idx, table, targets):
    B, T = idx.shape
    V = table.shape[0]
    N = B * T
    v_shift = (V - 1).bit_length()
    chunks = V // _LANES
    tm = min(_ROW_TILE, N)
    n_tiles = N // tm

    flat = (idx.reshape(N).astype(jnp.int32) * V
            + targets.reshape(N).astype(jnp.int32))
    flat3d = flat.reshape(N // tm, 1, tm)

    vb = min(256, V)
    d = pl.pallas_call(
        _stats_kernel,
        out_shape=jax.ShapeDtypeStruct((V, V), jnp.float32),
        grid=(V // vb,),
        in_specs=[pl.BlockSpec((vb, V), lambda i: (i, 0))],
        out_specs=pl.BlockSpec((vb, V), lambda i: (i, 0)),
        compiler_params=pltpu.CompilerParams(
            dimension_semantics=("parallel",),
            vmem_limit_bytes=32 * 1024 * 1024,
        ),
    )(table)

    table2 = table.reshape(V * chunks, _LANES)
    d_chunks = d.reshape(V * chunks, 1, _LANES)

    scratch = pltpu.VMEM(((chunks - 1) * _STRIDE + _GROUP, _LANES),
                         jnp.float32)
    logits, partials = pl.pallas_call(
        _make_main_kernel(tm, v_shift, chunks),
        out_shape=(
            jax.ShapeDtypeStruct((N, V), jnp.float32),
            jax.ShapeDtypeStruct((n_tiles, 1, _LANES), jnp.float32),
        ),
        grid=(n_tiles,),
        in_specs=[
            pl.BlockSpec((1, 1, tm), lambda i: (i, 0, 0),
                         memory_space=pltpu.SMEM),
            pl.BlockSpec((V * chunks, _LANES), lambda i: (0, 0)),
            pl.BlockSpec((V * chunks, 1, _LANES), lambda i: (0, 0, 0)),
        ],
        out_specs=(
            pl.BlockSpec((tm, V), lambda i: (i, 0)),
            pl.BlockSpec((1, 1, _LANES), lambda i: (i, 0, 0)),
        ),
        scratch_shapes=[scratch, scratch],
        compiler_params=pltpu.CompilerParams(
            dimension_semantics=("parallel",),
            vmem_limit_bytes=56 * 1024 * 1024,
        ),
        cost_estimate=pl.CostEstimate(
            flops=2 * N * V,
            transcendentals=0,
            bytes_accessed=N * V * 4 + 2 * V * V * 4 + N * 4,
        ),
    )(flat3d, table2, d_chunks)

    loss = jnp.sum(partials[:, 0, 0]) * (1.0 / N)
    return logits, loss


# speedup vs baseline: 1.0923x; 1.0923x over previous
import jax
import jax.numpy as jnp
from jax.experimental import pallas as pl
from jax.experimental.pallas import tpu as pltpu

_ROW_TILE = 256
_GROUP = 64
_STRIDE = _GROUP + 1
_LANES = 128
_N_ACC = 4
_STATS_BLK = 64


def _make_main_kernel(tm, v_shift, chunks):
    def _main_kernel(flat_ref, table_ref, out_ref, part_ref,
                     d2, ts_a, ts_b):
        @pl.when(pl.program_id(0) == 0)
        def _compute_d():
            for rb in range(0, table_ref.shape[0] // chunks, _STATS_BLK):
                x = table_ref[rb * chunks:(rb + _STATS_BLK) * chunks, :]
                x3 = x.reshape(_STATS_BLK, chunks, _LANES)
                m = jnp.max(x3, axis=(1, 2), keepdims=True)
                s = jnp.sum(jnp.exp(x3 - m), axis=(1, 2), keepdims=True)
                lse = jnp.log(s) + m
                d2[rb * chunks:(rb + _STATS_BLK) * chunks, :] = (
                    (lse - x3).reshape(_STATS_BLK * chunks, _LANES))

        accs = [jnp.zeros((1, _LANES), jnp.float32) for _ in range(_N_ACC)]
        for g in range(tm // _GROUP):
            ts = ts_a if g % 2 == 0 else ts_b
            for mi in range(_GROUP):
                m = g * _GROUP + mi
                f = flat_ref[0, 0, m]
                i16 = pl.multiple_of((f >> v_shift) * chunks,
                                     8 if chunks % 8 == 0 else chunks)
                slab = table_ref[pl.ds(i16, chunks), :]
                ts[mi:mi + (chunks - 1) * _STRIDE + 1:_STRIDE, :] = slab
                c8 = pl.multiple_of(((f >> 10) << 3), 8)
                blk = d2[pl.ds(c8, 8), :]
                blk = pltpu.roll(blk, -((f >> 7) & 7), axis=0)
                blk = pltpu.roll(blk, -(f & (_LANES - 1)), axis=1)
                accs[m % _N_ACC] = accs[m % _N_ACC] + blk[0:1, :]
            r0 = g * _GROUP
            for j in range(chunks):
                out_ref[r0:r0 + _GROUP, j * _LANES:(j + 1) * _LANES] = (
                    ts[j * _STRIDE:j * _STRIDE + _GROUP, :])
        acc = (accs[0] + accs[1]) + (accs[2] + accs[3])
        part_ref[...] = acc.reshape(1, 1, _LANES)
    return _main_kernel


def kernel(idx, table, targets):
    B, T = idx.shape
    V = table.shape[0]
    N = B * T
    v_shift = (V - 1).bit_length()
    chunks = V // _LANES
    tm = min(_ROW_TILE, N)
    n_tiles = N // tm

    flat = (idx.reshape(N).astype(jnp.int32) * V
            + targets.reshape(N).astype(jnp.int32))
    flat3d = flat.reshape(n_tiles, 1, tm)
    table2 = table.reshape(V * chunks, _LANES)

    scratch = pltpu.VMEM(((chunks - 1) * _STRIDE + _GROUP, _LANES),
                         jnp.float32)
    logits, partials = pl.pallas_call(
        _make_main_kernel(tm, v_shift, chunks),
        out_shape=(
            jax.ShapeDtypeStruct((N, V), jnp.float32),
            jax.ShapeDtypeStruct((n_tiles, 1, _LANES), jnp.float32),
        ),
        grid=(n_tiles,),
        in_specs=[
            pl.BlockSpec((1, 1, tm), lambda i: (i, 0, 0),
                         memory_space=pltpu.SMEM),
            pl.BlockSpec((V * chunks, _LANES), lambda i: (0, 0)),
        ],
        out_specs=(
            pl.BlockSpec((tm, V), lambda i: (i, 0)),
            pl.BlockSpec((1, 1, _LANES), lambda i: (i, 0, 0)),
        ),
        scratch_shapes=[pltpu.VMEM((V * chunks, _LANES), jnp.float32),
                        scratch, scratch],
        compiler_params=pltpu.CompilerParams(
            dimension_semantics=("arbitrary",),
            vmem_limit_bytes=56 * 1024 * 1024,
        ),
        cost_estimate=pl.CostEstimate(
            flops=2 * N * V,
            transcendentals=V * V,
            bytes_accessed=N * V * 4 + V * V * 4 + N * 4,
        ),
    )(flat3d, table2)

    loss = jnp.sum(partials[:, 0, 0]) * (1.0 / N)
    return logits, loss


# speedup vs baseline: 1.1340x; 1.0382x over previous
import jax
import jax.numpy as jnp
from jax.experimental import pallas as pl
from jax.experimental.pallas import tpu as pltpu

_ROW_TILE = 256
_GROUP = 64
_STRIDE = _GROUP + 1
_LANES = 128
_N_ACC = 4


def _make_main_kernel(tm, v_shift, chunks, v_total):
    def _main_kernel(flat_ref, table_ref, out_ref, part_ref,
                     t2, d2, ts_a, ts_b):
        @pl.when(pl.program_id(0) == 0)
        def _prep():
            def blk16(bb, _):
                for sub in range(16):
                    r0 = pl.multiple_of((bb * 16 + sub) * 8, 8)
                    x = table_ref[pl.ds(r0, 8), :]
                    m = jnp.max(x, axis=-1, keepdims=True)
                    s = jnp.sum(jnp.exp(x - m), axis=-1, keepdims=True)
                    d = (jnp.log(s) + m) - x
                    for v in range(chunks):
                        sl = pl.Slice(r0 * chunks + v, 8, chunks)
                        t2[sl, :] = x[:, v * _LANES:(v + 1) * _LANES]
                        d2[sl, :] = d[:, v * _LANES:(v + 1) * _LANES]
                return 0
            jax.lax.fori_loop(0, v_total // 128, blk16, 0)

        accs = [jnp.zeros((1, _LANES), jnp.float32) for _ in range(_N_ACC)]
        for g in range(tm // _GROUP):
            ts = ts_a if g % 2 == 0 else ts_b
            for mi in range(_GROUP):
                m = g * _GROUP + mi
                f = flat_ref[0, 0, m]
                i16 = pl.multiple_of((f >> v_shift) * chunks,
                                     8 if chunks % 8 == 0 else chunks)
                slab = t2[pl.ds(i16, chunks), :]
                ts[mi:mi + (chunks - 1) * _STRIDE + 1:_STRIDE, :] = slab
                c8 = pl.multiple_of(((f >> 10) << 3), 8)
                blk = d2[pl.ds(c8, 8), :]
                blk = pltpu.roll(blk, -((f >> 7) & 7), axis=0)
                blk = pltpu.roll(blk, -(f & (_LANES - 1)), axis=1)
                accs[m % _N_ACC] = accs[m % _N_ACC] + blk[0:1, :]
            r0 = g * _GROUP
            for j in range(chunks):
                out_ref[r0:r0 + _GROUP, j * _LANES:(j + 1) * _LANES] = (
                    ts[j * _STRIDE:j * _STRIDE + _GROUP, :])
        acc = (accs[0] + accs[1]) + (accs[2] + accs[3])
        part_ref[...] = acc.reshape(1, 1, _LANES)
    return _main_kernel


def kernel(idx, table, targets):
    B, T = idx.shape
    V = table.shape[0]
    N = B * T
    v_shift = (V - 1).bit_length()
    chunks = V // _LANES
    tm = min(_ROW_TILE, N)
    n_tiles = N // tm

    flat = (idx.reshape(N).astype(jnp.int32) * V
            + targets.reshape(N).astype(jnp.int32))
    flat3d = flat.reshape(n_tiles, 1, tm)

    scratch = pltpu.VMEM(((chunks - 1) * _STRIDE + _GROUP, _LANES),
                         jnp.float32)
    big = pltpu.VMEM((V * chunks, _LANES), jnp.float32)
    logits, partials = pl.pallas_call(
        _make_main_kernel(tm, v_shift, chunks, V),
        out_shape=(
            jax.ShapeDtypeStruct((N, V), jnp.float32),
            jax.ShapeDtypeStruct((n_tiles, 1, _LANES), jnp.float32),
        ),
        grid=(n_tiles,),
        in_specs=[
            pl.BlockSpec((1, 1, tm), lambda i: (i, 0, 0),
                         memory_space=pltpu.SMEM),
            pl.BlockSpec((V, V), lambda i: (0, 0)),
        ],
        out_specs=(
            pl.BlockSpec((tm, V), lambda i: (i, 0)),
            pl.BlockSpec((1, 1, _LANES), lambda i: (i, 0, 0)),
        ),
        scratch_shapes=[big, big, scratch, scratch],
        compiler_params=pltpu.CompilerParams(
            dimension_semantics=("arbitrary",),
            vmem_limit_bytes=58 * 1024 * 1024,
        ),
        cost_estimate=pl.CostEstimate(
            flops=2 * N * V,
            transcendentals=V * V,
            bytes_accessed=N * V * 4 + V * V * 4 + N * 4,
        ),
    )(flat3d, table)

    loss = jnp.sum(partials[:, 0, 0]) * (1.0 / N)
    return logits, loss
